# Initial kernel scaffold; baseline (speedup 1.0000x reference)
import jax
import jax.numpy as jnp
from jax.experimental import pallas as pl


def kernel(pred_logits, pred_boxes, pred_queries, siamese_logits, siamese_boxes, siamese_query, tgt_labels, tgt_boxes):
    raise NotImplementedError("write your pallas kernel here")


# speedup vs baseline: 10.7930x; 10.7930x over previous
import functools

import jax
import jax.numpy as jnp
from jax.experimental import pallas as pl

B, Q, C, D, T = 4, 900, 91, 256, 25
_HIGH = jax.lax.Precision.HIGHEST
_INTERPRET = False


def _match_body(lT, bT, labels_col, tbox):
    m = jnp.max(lT, axis=0, keepdims=True)
    e = jnp.exp(lT - m)
    s = jnp.sum(e, axis=0, keepdims=True)
    cls_iota = jax.lax.broadcasted_iota(jnp.int32, (T, C), 1)
    onehot = (labels_col == cls_iota).astype(jnp.float32)
    g = jax.lax.dot_general(onehot, e, (((1,), (0,)), ((), ())),
                            precision=_HIGH)
    costT = -2.0 * g / s
    cb = jnp.zeros((T, Q), jnp.float32)
    for k in range(4):
        cb = cb + jnp.abs(tbox[:, k:k + 1] - bT[k:k + 1, :])
    costT = costT + 5.0 * cb

    iota_q = jax.lax.broadcasted_iota(jnp.int32, (1, Q), 1)
    rowi = jax.lax.broadcasted_iota(jnp.int32, (T, Q), 0)
    avail = jnp.ones((1, Q), jnp.float32)
    S = jnp.zeros((T, Q), jnp.float32)
    for t in range(T):
        col = jnp.where(avail > 0.0, costT[t:t + 1, :], jnp.inf)
        mval = jnp.min(col)
        is_min = col == mval
        idx = jnp.min(jnp.where(is_min, iota_q, jnp.int32(2 ** 30)))
        sel = iota_q == idx
        S = jnp.where((rowi == t) & sel, 1.0, S)
        avail = jnp.where(sel, 0.0, avail)
    return S


def _body(lpT, bpT, qp, lsT, bsT, qs, lab, tb, out_ref):
    b = pl.program_id(0)
    labels_col = lab[0]
    tbox = tb[0]
    S1 = _match_body(lpT[0], bpT[0], labels_col, tbox)
    S2 = _match_body(lsT[0], bsT[0], labels_col, tbox)
    F1 = jax.lax.dot_general(S1, qp[0], (((1,), (0,)), ((), ())),
                             precision=_HIGH)
    F2 = jax.lax.dot_general(S2, qs[0], (((1,), (0,)), ((), ())),
                             precision=_HIGH)
    dots = jnp.sum(F1 * F2, axis=1, keepdims=True)
    n1 = jnp.maximum(jnp.sqrt(jnp.sum(F1 * F1, axis=1, keepdims=True)), 1e-8)
    n2 = jnp.maximum(jnp.sqrt(jnp.sum(F2 * F2, axis=1, keepdims=True)), 1e-8)
    csum = jnp.sum(dots / (n1 * n2), axis=0, keepdims=True)
    prev = jnp.where(b == 0, jnp.zeros((1, 1), jnp.float32), out_ref[:, :])
    tot = prev + csum
    out_ref[:, :] = jnp.where(b == B - 1, -tot / (B * T), tot)


@jax.jit
def kernel(pred_logits, pred_boxes, pred_queries, siamese_logits,
           siamese_boxes, siamese_query, tgt_labels, tgt_boxes):
    lpT = pred_logits.transpose(0, 2, 1)
    lsT = siamese_logits.transpose(0, 2, 1)
    bpT = pred_boxes.transpose(0, 2, 1)
    bsT = siamese_boxes.transpose(0, 2, 1)
    lab = tgt_labels.astype(jnp.int32).reshape(B, T, 1)
    spec = lambda shape: pl.BlockSpec((1,) + shape, lambda b: (b, 0, 0))
    out = pl.pallas_call(
        _body,
        grid=(B,),
        in_specs=[
            spec((C, Q)), spec((4, Q)), spec((Q, D)),
            spec((C, Q)), spec((4, Q)), spec((Q, D)),
            spec((T, 1)), spec((T, 4)),
        ],
        out_specs=pl.BlockSpec((1, 1), lambda b: (0, 0)),
        out_shape=jax.ShapeDtypeStruct((1, 1), jnp.float32),
        interpret=_INTERPRET,
    )(lpT, bpT, pred_queries, lsT, bsT, siamese_query, lab, tgt_boxes)
    return out.reshape(())


# speedup vs baseline: 29.4564x; 2.7292x over previous
import functools

import jax
import jax.numpy as jnp
from jax.experimental import pallas as pl

B, Q, C, D, T = 4, 900, 91, 256, 25
P = 2 * B
_HIGH = jax.lax.Precision.HIGHEST
_INTERPRET = False


def _body(lT, bT, lab, tb, qall, out_ref):
    m = jnp.max(lT[...], axis=1, keepdims=True)
    e = jnp.exp(lT[...] - m)
    s = jnp.sum(e, axis=1, keepdims=True)
    cls_iota = jax.lax.broadcasted_iota(jnp.int32, (P, T, C), 2)
    onehot = (lab[...] == cls_iota).astype(jnp.float32)
    g = jax.lax.dot_general(onehot, e, (((2,), (1,)), ((0,), (0,))),
                            precision=_HIGH)
    cost = -2.0 * g / s
    for k in range(4):
        cost = cost + 5.0 * jnp.abs(tb[:, :, k:k + 1] - bT[:, k:k + 1, :])

    iota_q = jax.lax.broadcasted_iota(jnp.int32, (P, Q), 1)
    tcol = jax.lax.broadcasted_iota(jnp.int32, (P, T), 1)
    avail = jnp.ones((P, Q), jnp.float32)
    I = jnp.zeros((P, T), jnp.int32)
    for t in range(T):
        col = cost[:, t, :].reshape(P, Q)
        col = jnp.where(avail > 0.0, col, jnp.inf)
        mval = jnp.min(col, axis=1, keepdims=True)
        idx = jnp.min(jnp.where(col == mval, iota_q, jnp.int32(2 ** 30)),
                      axis=1, keepdims=True)
        avail = jnp.where(iota_q == idx, 0.0, avail)
        I = jnp.where(tcol == t, idx, I)

    qiota = jax.lax.broadcasted_iota(jnp.int32, (Q, T), 0)
    F = []
    for p in range(P):
        S_T = (qiota == I[p:p + 1, :]).astype(jnp.float32)
        F.append(jax.lax.dot_general(S_T, qall[p], (((0,), (0,)), ((), ())),
                                     precision=_HIGH))
    total = jnp.zeros((1, 1), jnp.float32)
    for b in range(B):
        F1, F2 = F[b], F[b + B]
        dots = jnp.sum(F1 * F2, axis=1, keepdims=True)
        n1 = jnp.maximum(jnp.sqrt(jnp.sum(F1 * F1, 1, keepdims=True)), 1e-8)
        n2 = jnp.maximum(jnp.sqrt(jnp.sum(F2 * F2, 1, keepdims=True)), 1e-8)
        total = total + jnp.sum(dots / (n1 * n2), axis=0, keepdims=True)
    out_ref[:, :] = -total / (B * T)


@jax.jit
def kernel(pred_logits, pred_boxes, pred_queries, siamese_logits,
           siamese_boxes, siamese_query, tgt_labels, tgt_boxes):
    lT = jnp.concatenate([pred_logits, siamese_logits], 0).transpose(0, 2, 1)
    bT = jnp.concatenate([pred_boxes, siamese_boxes], 0).transpose(0, 2, 1)
    qall = jnp.concatenate([pred_queries, siamese_query], 0)
    lab = jnp.tile(tgt_labels.astype(jnp.int32).reshape(B, T, 1), (2, 1, 1))
    tb = jnp.tile(tgt_boxes, (2, 1, 1))
    out = pl.pallas_call(
        _body,
        out_shape=jax.ShapeDtypeStruct((1, 1), jnp.float32),
        interpret=_INTERPRET,
    )(lT, bT, lab, tb, qall)
    return out.reshape(())
